# baseline (speedup 1.0000x reference)
import functools

import jax
import jax.numpy as jnp
from jax import lax
from jax.experimental import pallas as pl
from jax.experimental.pallas import tpu as pltpu
from jax.experimental.pallas import tpu_sc as plsc

NQ = 10000
NS = 100000
E = 320000
D = 128
H = 4
DH = D // H
LEN_DIM = 32
CTX_DIM = 32
FC = 64
R_CUTOFF = 10.0

BQ = 128
NQB = (NQ + BQ - 1) // BQ
NQP = NQB * BQ
BE = 2048
GW = 256
E_PAD = 327680
TW = 144
BS = 1000

_dot = functools.partial(
    lax.dot_general,
    precision=lax.Precision.DEFAULT,
    preferred_element_type=jnp.float32,
)

_SIN_C = (0.9999997069582272, -0.16666577198087604, 0.008332557998374063,
          -0.0001981257223755738, 2.704047331301695e-06,
          -2.053408004778425e-08)


def _fast_sin(a):
    m = jnp.round(a * (1.0 / (2.0 * jnp.pi)))
    x = (a - m * 6.28125) - m * 0.0019353071795864769
    x2 = x * x
    p = jnp.float32(_SIN_C[5])
    for k in (4, 3, 2, 1, 0):
        p = p * x2 + jnp.float32(_SIN_C[k])
    return p * x


def _mm(a, b):
    return _dot(a, b, dimension_numbers=(((1,), (0,)), ((), ())))


def _table_body(x_ref, p_ref, wv_ref, o_ref):
    vperm = _mm(x_ref[...], wv_ref[...])
    a = vperm[:, 0:64].astype(jnp.bfloat16)
    b = vperm[:, 64:128].astype(jnp.bfloat16)
    au = lax.bitcast_convert_type(a, jnp.uint16).astype(jnp.uint32)
    bu = lax.bitcast_convert_type(b, jnp.uint16).astype(jnp.uint32)
    packed = au | (bu << 16)
    o_ref[:, 0:64] = lax.bitcast_convert_type(packed, jnp.int32)
    o_ref[:, 64:80] = p_ref[...]
    o_ref[:, 80:128] = jnp.zeros((BS, 48), jnp.int32)


def _build_table(src_x, pos_i32, W_v_perm):
    return pl.pallas_call(
        _table_body,
        grid=(NS // BS,),
        in_specs=[
            pl.BlockSpec((BS, D), lambda i: (i, 0)),
            pl.BlockSpec((BS, 16), lambda i: (i, 0)),
            pl.BlockSpec((D, D), lambda i: (0, 0)),
        ],
        out_specs=pl.BlockSpec((BS, D), lambda i: (i, 0)),
        out_shape=jax.ShapeDtypeStruct((NS, D), jnp.int32),
    )(src_x, pos_i32, W_v_perm)


def _sc_gather(table, idx):
    idx2 = idx.reshape(1, E_PAD)
    mesh = plsc.VectorSubcoreMesh(
        core_axis_name="core", subcore_axis_name="subcore"
    )

    @functools.partial(
        pl.kernel,
        out_type=jax.ShapeDtypeStruct((E_PAD, D), jnp.int32),
        mesh=mesh,
    )
    def gather_kernel(tbl_hbm, i_hbm, o_hbm):
        def body(i_vmem, o_vmem):
            pltpu.sync_copy(tbl_hbm.at[i_vmem.at[0]], o_vmem)

        pltpu.emit_pipeline(
            body,
            grid=(E_PAD // GW,),
            in_specs=[pl.BlockSpec((1, GW), index_map=lambda i: (0, i))],
            out_specs=[pl.BlockSpec((GW, D), index_map=lambda i: (i, 0))],
            core_axis_name=("core", "subcore"),
            dimension_semantics=(pltpu.PARALLEL,),
        )(i_hbm, o_hbm)

    return gather_kernel(table, idx2)


def _main_body(estart, t_hbm, ed_hbm, qx, qpos, ctx,
               Wq, W1a, W1b, b1, W2, b2, Wm, wa, G, GT, Wout, bout,
               out, tbuf, dbuf, sem_t, sem_d):
    j = pl.program_id(0)
    s0 = (estart[j] // 16) * 16
    n = estart[j + 1] - s0
    qproj = _mm(qx[...], Wq[...])
    ctxw = _mm(ctx[...], W1b[...])
    n_chunks = (n + BE - 1) // BE

    def _copies(c):
        slot = lax.rem(c, 2)
        base = s0 + c * BE
        return (
            pltpu.make_async_copy(
                t_hbm.at[pl.ds(base, BE)], tbuf.at[slot], sem_t.at[slot]),
            pltpu.make_async_copy(
                ed_hbm.at[pl.ds(base, BE)], dbuf.at[slot], sem_d.at[slot]),
        )

    @pl.when(n_chunks > 0)
    def _():
        for cp in _copies(0):
            cp.start()

    def body(c, carry):
        num, den = carry
        slot = lax.rem(c, 2)
        for cp in _copies(c):
            cp.wait()

        @pl.when(c + 1 < n_chunks)
        def _():
            for cp in _copies(c + 1):
                cp.start()

        lid = dbuf[slot] - j * BQ
        onehot = (lid == lax.broadcasted_iota(jnp.int32, (BE, BQ), 1)
                  ).astype(jnp.float32)

        trow = tbuf[slot]
        pk = lax.bitcast_convert_type(trow[:, 0:64], jnp.uint32)
        vlo = lax.bitcast_convert_type(pk << 16, jnp.float32)
        vhi = lax.bitcast_convert_type(pk & jnp.uint32(0xFFFF0000),
                                       jnp.float32)
        vrow = jnp.concatenate([vlo, vhi], axis=1)
        spos = trow[:, 64:80]
        posq = _mm(onehot, qpos[...])
        rel = posq - spos
        r = jnp.sqrt(jnp.sum(rel * rel, axis=1, keepdims=True) + 1e-12)
        karr = (lax.broadcasted_iota(jnp.int32, (BE, LEN_DIM), 1) + 1
                ).astype(jnp.float32)
        bes = _fast_sin(karr * ((jnp.pi / R_CUTOFF) * r)) * (
            jnp.sqrt(2.0 / R_CUTOFF) / (r + 1e-6))

        pre1 = _mm(bes, W1a[...]) + _mm(onehot, ctxw) + b1[...]
        h = pre1 * jax.nn.sigmoid(pre1)
        pre2 = _mm(h, W2[...]) + b2[...]
        w_e = pre2 * jax.nn.sigmoid(pre2)
        gate = _mm(w_e, Wm[...])
        vh = vrow * gate
        qg = _mm(onehot, qproj)
        f = qg + vh
        f = jnp.where(f >= 0.0, f, 0.2 * f)
        lg = _mm(f * wa[...], G[...])
        lg = jnp.clip(lg, -80.0, 80.0)
        ex = jnp.exp(lg)
        exb = _mm(ex, GT[...])
        num = num + _dot(onehot, exb * vh,
                         dimension_numbers=(((0,), (0,)), ((), ())))
        den = den + _dot(onehot, ex,
                         dimension_numbers=(((0,), (0,)), ((), ())))
        return num, den

    num0 = jnp.zeros((BQ, 128), jnp.float32)
    den0 = jnp.zeros((BQ, 8), jnp.float32)
    num, den = lax.fori_loop(0, n_chunks, body, (num0, den0))
    denb = _mm(den, GT[...])
    outh = num / (denb + 1e-9)
    out[...] = _mm(outh, Wout[...]) + bout[...] + qx[...]


def _run_main(estart, t_g, ed_col, qx_p, qpos_p, ctx_p,
              Wq, W1a, W1b, b1, W2, b2, Wm, wa, G, GT, Wout, bout):
    grid_spec = pltpu.PrefetchScalarGridSpec(
        num_scalar_prefetch=1,
        grid=(NQB,),
        in_specs=[
            pl.BlockSpec(memory_space=pl.ANY),
            pl.BlockSpec(memory_space=pl.ANY),
            pl.BlockSpec((BQ, D), lambda j, es: (j, 0)),
            pl.BlockSpec((BQ, 16), lambda j, es: (j, 0)),
            pl.BlockSpec((BQ, CTX_DIM), lambda j, es: (j, 0)),
            pl.BlockSpec((D, D), lambda j, es: (0, 0)),
            pl.BlockSpec((LEN_DIM, FC), lambda j, es: (0, 0)),
            pl.BlockSpec((CTX_DIM, FC), lambda j, es: (0, 0)),
            pl.BlockSpec((1, FC), lambda j, es: (0, 0)),
            pl.BlockSpec((FC, FC), lambda j, es: (0, 0)),
            pl.BlockSpec((1, FC), lambda j, es: (0, 0)),
            pl.BlockSpec((FC, D), lambda j, es: (0, 0)),
            pl.BlockSpec((1, D), lambda j, es: (0, 0)),
            pl.BlockSpec((D, 8), lambda j, es: (0, 0)),
            pl.BlockSpec((8, D), lambda j, es: (0, 0)),
            pl.BlockSpec((D, D), lambda j, es: (0, 0)),
            pl.BlockSpec((1, D), lambda j, es: (0, 0)),
        ],
        out_specs=pl.BlockSpec((BQ, D), lambda j, es: (j, 0)),
        scratch_shapes=[
            pltpu.VMEM((2, BE, D), jnp.float32),
            pltpu.VMEM((2, BE, 1), jnp.int32),
            pltpu.SemaphoreType.DMA((2,)),
            pltpu.SemaphoreType.DMA((2,)),
        ],
    )
    return pl.pallas_call(
        _main_body,
        grid_spec=grid_spec,
        out_shape=jax.ShapeDtypeStruct((NQP, D), jnp.float32),
    )(estart, t_g, ed_col, qx_p, qpos_p, ctx_p,
      Wq, W1a, W1b, b1, W2, b2, Wm, wa, G, GT, Wout, bout)


def kernel(query_x, query_pos, src_x, src_pos, context_emb, edge_src,
           edge_dst, W_q, W_v, W_fc1, b_fc1, W_fc2, b_fc2, W_mod, w_alpha,
           W_out, b_out):
    f32 = jnp.float32
    edge_src = edge_src.astype(jnp.int32)
    edge_dst = edge_dst.astype(jnp.int32)

    pos_i32 = lax.bitcast_convert_type(
        jnp.pad(src_pos.astype(f32), ((0, 0), (0, 13))), jnp.int32)
    qpos_p = jnp.pad(query_pos.astype(f32), ((0, NQP - NQ), (0, 13)))
    qx_p = jnp.pad(query_x.astype(f32), ((0, NQP - NQ), (0, 0)))
    ctx_p = jnp.pad(context_emb.astype(f32), ((0, NQP - NQ), (0, 0)))
    es_p = jnp.pad(edge_src, (0, E_PAD - E))
    ed_p = jnp.pad(edge_dst, (0, E_PAD - E), constant_values=NQ)
    ed_col = ed_p.reshape(E_PAD, 1)
    bounds = (jnp.arange(NQB + 1, dtype=jnp.int32) * BQ)
    estart = jnp.searchsorted(edge_dst, bounds, side="left").astype(jnp.int32)

    W1a = W_fc1[:LEN_DIM]
    W1b = W_fc1[LEN_DIM:]
    b1 = b_fc1.reshape(1, FC).astype(f32)
    b2 = b_fc2.reshape(1, FC).astype(f32)
    bout = b_out.reshape(1, D).astype(f32)

    perm = jnp.concatenate([jnp.arange(0, D, 2), jnp.arange(1, D, 2)])
    wa = w_alpha.reshape(1, D).astype(f32)[:, perm]
    G = (lax.broadcasted_iota(jnp.int32, (D, 8), 0) // DH
         == lax.broadcasted_iota(jnp.int32, (D, 8), 1)).astype(f32)[perm]
    GT = G.T
    Wq_p = W_q.astype(f32)[:, perm]
    Wm_p = W_mod.astype(f32)[:, perm]
    Wout_p = W_out.astype(f32)[perm, :]

    table = _build_table(src_x.astype(f32), pos_i32,
                         W_v.astype(f32)[:, perm])
    t_g = _sc_gather(table, es_p)
    t_gf = lax.bitcast_convert_type(t_g, jnp.float32)
    out_p = _run_main(estart, t_gf, ed_col, qx_p, qpos_p, ctx_p,
                      Wq_p, W1a, W1b, b1, W_fc2.astype(f32), b2,
                      Wm_p, wa, G, GT, Wout_p, bout)
    return out_p[:NQ]


# speedup vs baseline: 1.3485x; 1.3485x over previous
import functools

import jax
import jax.numpy as jnp
from jax import lax
from jax.experimental import pallas as pl
from jax.experimental.pallas import tpu as pltpu
from jax.experimental.pallas import tpu_sc as plsc

NQ = 10000
NS = 100000
E = 320000
D = 128
H = 4
DH = D // H
LEN_DIM = 32
CTX_DIM = 32
FC = 64
R_CUTOFF = 10.0

BQ = 128
NQB = (NQ + BQ - 1) // BQ
NQP = NQB * BQ
BE = 2048
GW = 128
E_PAD = 323584
TW = 144
BS = 1000

_dot = functools.partial(
    lax.dot_general,
    precision=lax.Precision.DEFAULT,
    preferred_element_type=jnp.float32,
)

_SIN_C = (0.9999997069582272, -0.16666577198087604, 0.008332557998374063,
          -0.0001981257223755738, 2.704047331301695e-06,
          -2.053408004778425e-08)


def _fast_sin(a):
    m = jnp.round(a * (1.0 / (2.0 * jnp.pi)))
    x = (a - m * 6.28125) - m * 0.0019353071795864769
    x2 = x * x
    p = jnp.float32(_SIN_C[5])
    for k in (4, 3, 2, 1, 0):
        p = p * x2 + jnp.float32(_SIN_C[k])
    return p * x


def _mm(a, b):
    return _dot(a, b, dimension_numbers=(((1,), (0,)), ((), ())))


def _table_body(x_ref, p_ref, wv_ref, o_ref):
    vperm = _mm(x_ref[...], wv_ref[...])
    a = vperm[:, 0:64].astype(jnp.bfloat16)
    b = vperm[:, 64:128].astype(jnp.bfloat16)
    au = lax.bitcast_convert_type(a, jnp.uint16).astype(jnp.uint32)
    bu = lax.bitcast_convert_type(b, jnp.uint16).astype(jnp.uint32)
    packed = au | (bu << 16)
    o_ref[:, 0:64] = lax.bitcast_convert_type(packed, jnp.int32)
    o_ref[:, 64:80] = p_ref[...]
    o_ref[:, 80:128] = jnp.zeros((BS, 48), jnp.int32)


def _build_table(src_x, pos_i32, W_v_perm):
    return pl.pallas_call(
        _table_body,
        grid=(NS // BS,),
        in_specs=[
            pl.BlockSpec((BS, D), lambda i: (i, 0)),
            pl.BlockSpec((BS, 16), lambda i: (i, 0)),
            pl.BlockSpec((D, D), lambda i: (0, 0)),
        ],
        out_specs=pl.BlockSpec((BS, D), lambda i: (i, 0)),
        out_shape=jax.ShapeDtypeStruct((NS, D), jnp.int32),
        compiler_params=pltpu.CompilerParams(
            dimension_semantics=("parallel",)),
    )(src_x, pos_i32, W_v_perm)


def _sc_gather(table, idx):
    idx2 = idx.reshape(1, E_PAD)
    mesh = plsc.VectorSubcoreMesh(
        core_axis_name="core", subcore_axis_name="subcore"
    )

    @functools.partial(
        pl.kernel,
        out_type=jax.ShapeDtypeStruct((E_PAD, D), jnp.int32),
        mesh=mesh,
    )
    def gather_kernel(tbl_hbm, i_hbm, o_hbm):
        def body(i_vmem, o_vmem):
            pltpu.sync_copy(tbl_hbm.at[i_vmem.at[0]], o_vmem)

        pltpu.emit_pipeline(
            body,
            grid=(E_PAD // GW,),
            in_specs=[pl.BlockSpec((1, GW), index_map=lambda i: (0, i))],
            out_specs=[pl.BlockSpec((GW, D), index_map=lambda i: (i, 0))],
            core_axis_name=("core", "subcore"),
            dimension_semantics=(pltpu.PARALLEL,),
        )(i_hbm, o_hbm)

    return gather_kernel(table, idx2)


def _main_body(estart, t_hbm, ed_hbm, qx, qpos, ctx,
               Wq, W1a, W1b, b1, W2, b2, Wm, wa, G, GT, Wout, bout,
               lmask, Slo, Shi, ones16,
               out, tbuf, dbuf, sem_t, sem_d):
    j = pl.program_id(0)
    s0 = (estart[j] // 16) * 16
    n = estart[j + 1] - s0
    qproj = _mm(qx[...], Wq[...])
    ctxw = _mm(ctx[...], W1b[...])
    n_chunks = (n + BE - 1) // BE

    def _copies(c):
        slot = lax.rem(c, 2)
        base = s0 + c * BE
        return (
            pltpu.make_async_copy(
                t_hbm.at[pl.ds(base, BE)], tbuf.at[slot], sem_t.at[slot]),
            pltpu.make_async_copy(
                ed_hbm.at[pl.ds(base, BE)], dbuf.at[slot], sem_d.at[slot]),
        )

    @pl.when(n_chunks > 0)
    def _():
        for cp in _copies(0):
            cp.start()

    def body(c, carry):
        num, den = carry
        slot = lax.rem(c, 2)
        for cp in _copies(c):
            cp.wait()

        @pl.when(c + 1 < n_chunks)
        def _():
            for cp in _copies(c + 1):
                cp.start()

        lid = dbuf[slot] - j * BQ
        onehot = (lid == lax.broadcasted_iota(jnp.int32, (BE, BQ), 1)
                  ).astype(jnp.float32)

        trow = tbuf[slot]
        pk = lax.bitcast_convert_type(trow, jnp.uint32)
        vlo = lax.bitcast_convert_type((pk << 16) & lmask[...],
                                       jnp.float32)
        vhi = lax.bitcast_convert_type(pk & jnp.uint32(0xFFFF0000),
                                       jnp.float32)
        vrow = _mm(vlo, Slo[...]) + _mm(vhi, Shi[...])
        spos = trow[:, 64:80]
        posq = _mm(onehot, qpos[...])
        rel = posq - spos
        rel2 = rel * rel
        r = jnp.sqrt(_mm(rel2, ones16[...])[:, 0:1] + 1e-12)
        karr = (lax.broadcasted_iota(jnp.int32, (BE, LEN_DIM), 1) + 1
                ).astype(jnp.float32)
        bes = _fast_sin(karr * ((jnp.pi / R_CUTOFF) * r)) * (
            jnp.sqrt(2.0 / R_CUTOFF) / (r + 1e-6))

        pre1 = _mm(bes, W1a[...]) + _mm(onehot, ctxw) + b1[...]
        h = pre1 * jax.nn.sigmoid(pre1)
        pre2 = _mm(h, W2[...]) + b2[...]
        w_e = pre2 * jax.nn.sigmoid(pre2)
        gate = _mm(w_e, Wm[...])
        vh = vrow * gate
        qg = _mm(onehot, qproj)
        f = qg + vh
        f = jnp.where(f >= 0.0, f, 0.2 * f)
        lg = _mm(f * wa[...], G[...])
        lg = jnp.clip(lg, -80.0, 80.0)
        ex = jnp.exp(lg)
        exb = _mm(ex, GT[...])
        num = num + _dot(onehot, exb * vh,
                         dimension_numbers=(((0,), (0,)), ((), ())))
        den = den + _dot(onehot, ex,
                         dimension_numbers=(((0,), (0,)), ((), ())))
        return num, den

    num0 = jnp.zeros((BQ, 128), jnp.float32)
    den0 = jnp.zeros((BQ, 8), jnp.float32)
    num, den = lax.fori_loop(0, n_chunks, body, (num0, den0))
    denb = _mm(den, GT[...])
    outh = num / (denb + 1e-9)
    out[...] = _mm(outh, Wout[...]) + bout[...] + qx[...]


def _run_main(estart, t_g, ed_col, qx_p, qpos_p, ctx_p,
              Wq, W1a, W1b, b1, W2, b2, Wm, wa, G, GT, Wout, bout,
              lmask, Slo, Shi, ones16):
    grid_spec = pltpu.PrefetchScalarGridSpec(
        num_scalar_prefetch=1,
        grid=(NQB,),
        in_specs=[
            pl.BlockSpec(memory_space=pl.ANY),
            pl.BlockSpec(memory_space=pl.ANY),
            pl.BlockSpec((BQ, D), lambda j, es: (j, 0)),
            pl.BlockSpec((BQ, 16), lambda j, es: (j, 0)),
            pl.BlockSpec((BQ, CTX_DIM), lambda j, es: (j, 0)),
            pl.BlockSpec((D, D), lambda j, es: (0, 0)),
            pl.BlockSpec((LEN_DIM, FC), lambda j, es: (0, 0)),
            pl.BlockSpec((CTX_DIM, FC), lambda j, es: (0, 0)),
            pl.BlockSpec((1, FC), lambda j, es: (0, 0)),
            pl.BlockSpec((FC, FC), lambda j, es: (0, 0)),
            pl.BlockSpec((1, FC), lambda j, es: (0, 0)),
            pl.BlockSpec((FC, D), lambda j, es: (0, 0)),
            pl.BlockSpec((1, D), lambda j, es: (0, 0)),
            pl.BlockSpec((D, 8), lambda j, es: (0, 0)),
            pl.BlockSpec((8, D), lambda j, es: (0, 0)),
            pl.BlockSpec((D, D), lambda j, es: (0, 0)),
            pl.BlockSpec((1, D), lambda j, es: (0, 0)),
            pl.BlockSpec((1, D), lambda j, es: (0, 0)),
            pl.BlockSpec((D, D), lambda j, es: (0, 0)),
            pl.BlockSpec((D, D), lambda j, es: (0, 0)),
            pl.BlockSpec((16, 8), lambda j, es: (0, 0)),
        ],
        out_specs=pl.BlockSpec((BQ, D), lambda j, es: (j, 0)),
        scratch_shapes=[
            pltpu.VMEM((2, BE, D), jnp.float32),
            pltpu.VMEM((2, BE, 1), jnp.int32),
            pltpu.SemaphoreType.DMA((2,)),
            pltpu.SemaphoreType.DMA((2,)),
        ],
    )
    return pl.pallas_call(
        _main_body,
        grid_spec=grid_spec,
        out_shape=jax.ShapeDtypeStruct((NQP, D), jnp.float32),
        compiler_params=pltpu.CompilerParams(
            dimension_semantics=("parallel",)),
    )(estart, t_g, ed_col, qx_p, qpos_p, ctx_p,
      Wq, W1a, W1b, b1, W2, b2, Wm, wa, G, GT, Wout, bout,
      lmask, Slo, Shi, ones16)


def kernel(query_x, query_pos, src_x, src_pos, context_emb, edge_src,
           edge_dst, W_q, W_v, W_fc1, b_fc1, W_fc2, b_fc2, W_mod, w_alpha,
           W_out, b_out):
    f32 = jnp.float32
    edge_src = edge_src.astype(jnp.int32)
    edge_dst = edge_dst.astype(jnp.int32)

    pos_i32 = lax.bitcast_convert_type(
        jnp.pad(src_pos.astype(f32), ((0, 0), (0, 13))), jnp.int32)
    qpos_p = jnp.pad(query_pos.astype(f32), ((0, NQP - NQ), (0, 13)))
    qx_p = jnp.pad(query_x.astype(f32), ((0, NQP - NQ), (0, 0)))
    ctx_p = jnp.pad(context_emb.astype(f32), ((0, NQP - NQ), (0, 0)))
    es_p = jnp.pad(edge_src, (0, E_PAD - E))
    ed_p = jnp.pad(edge_dst, (0, E_PAD - E), constant_values=NQ)
    ed_col = ed_p.reshape(E_PAD, 1)
    bounds = (jnp.arange(NQB + 1, dtype=jnp.int32) * BQ)
    estart = jnp.searchsorted(edge_dst, bounds, side="left").astype(jnp.int32)

    W1a = W_fc1[:LEN_DIM]
    W1b = W_fc1[LEN_DIM:]
    b1 = b_fc1.reshape(1, FC).astype(f32)
    b2 = b_fc2.reshape(1, FC).astype(f32)
    bout = b_out.reshape(1, D).astype(f32)

    perm = jnp.concatenate([jnp.arange(0, D, 2), jnp.arange(1, D, 2)])
    wa = w_alpha.reshape(1, D).astype(f32)[:, perm]
    G = (lax.broadcasted_iota(jnp.int32, (D, 8), 0) // DH
         == lax.broadcasted_iota(jnp.int32, (D, 8), 1)).astype(f32)[perm]
    GT = G.T
    Wq_p = W_q.astype(f32)[:, perm]
    Wm_p = W_mod.astype(f32)[:, perm]
    Wout_p = W_out.astype(f32)[perm, :]

    table = _build_table(src_x.astype(f32), pos_i32,
                         W_v.astype(f32)[:, perm])
    t_g = _sc_gather(table, es_p)
    lanes = jnp.arange(D)
    lmask = jnp.where(lanes < 64, jnp.uint32(0xFFFFFFFF),
                      jnp.uint32(0)).reshape(1, D)
    Slo = ((lanes[:, None] == lanes[None, :]) & (lanes[:, None] < 64)
           ).astype(f32)
    Shi = (lanes[None, :] == lanes[:, None] + 64).astype(f32)
    ones16 = jnp.ones((16, 8), f32)

    t_gf = lax.bitcast_convert_type(t_g, jnp.float32)
    out_p = _run_main(estart, t_gf, ed_col, qx_p, qpos_p, ctx_p,
                      Wq_p, W1a, W1b, b1, W_fc2.astype(f32), b2,
                      Wm_p, wa, G, GT, Wout_p, bout,
                      lmask, Slo, Shi, ones16)
    return out_p[:NQ]
